# Initial kernel scaffold; baseline (speedup 1.0000x reference)
import jax
import jax.numpy as jnp
from jax.experimental import pallas as pl


def kernel(sequence_bucket_inputs, tables):
    raise NotImplementedError("write your pallas kernel here")


# speedup vs baseline: 7.3451x; 7.3451x over previous
import functools

import numpy as np
import jax
import jax.numpy as jnp
from jax import lax
from jax.experimental import pallas as pl
from jax.experimental.pallas import tpu as pltpu
from jax.experimental.pallas import tpu_sc as plsc

MAX_SLOT = 20
TIME_STEPS = 20
VALID_SLOTS = tuple(s for s in range(MAX_SLOT) if s not in (0, 5))
NUM_SLOTS = len(VALID_SLOTS)
NUM_EMB = 1002
EMB_DIM = 32
BATCH = 1024

NUM_WORKERS = 32
PAIR = 2
ROWS_PER_B = TIME_STEPS * NUM_SLOTS
ROWS_PER_PAIR = ROWS_PER_B * PAIR
RAW_PER_PAIR = PAIR * TIME_STEPS * MAX_SLOT
CHUNK = 120
NUM_CHUNKS = ROWS_PER_PAIR // CHUNK
NUM_VECS = ROWS_PER_PAIR // 16
PAIRS_PER_W = (BATCH // PAIR) // NUM_WORKERS

_POS = np.array(
    [p * TIME_STEPS * MAX_SLOT + t * MAX_SLOT + s
     for p in range(PAIR) for t in range(TIME_STEPS) for s in VALID_SLOTS],
    dtype=np.int32)
_OFF = np.array(
    [(t * NUM_SLOTS + j) * NUM_EMB
     for p in range(PAIR) for t in range(TIME_STEPS) for j in range(NUM_SLOTS)],
    dtype=np.int32)


def _sc_body(tab_hbm, raw_hbm, pos_hbm, off_hbm, out_hbm,
             pos_v, off_v, raw_v, idx_v, rows_v, sem):
    wid = lax.axis_index("s") * 2 + lax.axis_index("c")
    base = wid * PAIRS_PER_W
    pltpu.sync_copy(pos_hbm, pos_v)
    pltpu.sync_copy(off_hbm, off_v)

    def body(i, carry):
        b2 = base + i
        pltpu.sync_copy(raw_hbm.at[b2], raw_v)
        for c in range(NUM_VECS):
            pv = pos_v[pl.ds(c * 16, 16)]
            ov = off_v[pl.ds(c * 16, 16)]
            vals = plsc.load_gather(raw_v, [pv])
            idx_v[pl.ds(c * 16, 16)] = vals + ov
        copies = [
            pltpu.async_copy(
                tab_hbm.at[idx_v.at[pl.ds(j * CHUNK, CHUNK)]],
                rows_v.at[pl.ds(j * CHUNK, CHUNK)], sem)
            for j in range(NUM_CHUNKS)
        ]
        for cp in copies:
            cp.wait()
        pltpu.sync_copy(rows_v, out_hbm.at[b2])
        return carry

    lax.fori_loop(0, PAIRS_PER_W, body, 0)


@jax.jit
def _run(tab, raw, pos, off):
    mesh = plsc.VectorSubcoreMesh(core_axis_name="c", subcore_axis_name="s")
    f = functools.partial(
        pl.kernel,
        mesh=mesh,
        compiler_params=pltpu.CompilerParams(
            needs_layout_passes=False, use_tc_tiling_on_sc=False),
        out_type=jax.ShapeDtypeStruct(
            (BATCH // PAIR, ROWS_PER_PAIR, EMB_DIM), jnp.float32),
        scratch_types=[
            pltpu.VMEM((ROWS_PER_PAIR,), jnp.int32),
            pltpu.VMEM((ROWS_PER_PAIR,), jnp.int32),
            pltpu.VMEM((RAW_PER_PAIR,), jnp.int32),
            pltpu.VMEM((ROWS_PER_PAIR,), jnp.int32),
            pltpu.VMEM((ROWS_PER_PAIR, EMB_DIM), jnp.float32),
            pltpu.SemaphoreType.DMA,
        ],
    )(_sc_body)
    return f(tab, raw, pos, off)


def kernel(sequence_bucket_inputs, tables):
    raw = sequence_bucket_inputs.reshape(BATCH // PAIR, RAW_PER_PAIR)
    tab = tables.reshape(TIME_STEPS * NUM_SLOTS * NUM_EMB, EMB_DIM)
    out = _run(tab, raw, jnp.asarray(_POS), jnp.asarray(_OFF))
    return out.reshape(BATCH, TIME_STEPS, NUM_SLOTS * EMB_DIM)


# speedup vs baseline: 7.7029x; 1.0487x over previous
import functools

import numpy as np
import jax
import jax.numpy as jnp
from jax import lax
from jax.experimental import pallas as pl
from jax.experimental.pallas import tpu as pltpu
from jax.experimental.pallas import tpu_sc as plsc

MAX_SLOT = 20
TIME_STEPS = 20
VALID_SLOTS = tuple(s for s in range(MAX_SLOT) if s not in (0, 5))
NUM_SLOTS = len(VALID_SLOTS)
NUM_EMB = 1002
EMB_DIM = 32
BATCH = 1024

NUM_WORKERS = 32
PAIR = 2
ROWS_PER_B = TIME_STEPS * NUM_SLOTS
ROWS_PER_PAIR = ROWS_PER_B * PAIR
RAW_PER_PAIR = PAIR * TIME_STEPS * MAX_SLOT
CHUNK = 720
NUM_CHUNKS = ROWS_PER_PAIR // CHUNK
NUM_VECS = ROWS_PER_PAIR // 16
NUM_PAIRS = BATCH // PAIR
PAIRS_PER_W = NUM_PAIRS // NUM_WORKERS

_POS = np.array(
    [p * TIME_STEPS * MAX_SLOT + t * MAX_SLOT + s
     for p in range(PAIR) for t in range(TIME_STEPS) for s in VALID_SLOTS],
    dtype=np.int32)
_OFF = np.array(
    [(t * NUM_SLOTS + j) * NUM_EMB
     for p in range(PAIR) for t in range(TIME_STEPS) for j in range(NUM_SLOTS)],
    dtype=np.int32)


def _sc_body(tab_hbm, raw_hbm, pos_hbm, off_hbm, out_hbm,
             pos_v, off_v, raw0, raw1, idx0, idx1, rows0, rows1,
             sem_r0, sem_r1, sem_g0, sem_g1, sem_o0, sem_o1):
    wid = lax.axis_index("s") * 2 + lax.axis_index("c")
    base = wid * PAIRS_PER_W
    raws, idxs, rows = (raw0, raw1), (idx0, idx1), (rows0, rows1)
    sem_r, sem_g, sem_o = (sem_r0, sem_r1), (sem_g0, sem_g1), (sem_o0, sem_o1)

    pltpu.sync_copy(pos_hbm, pos_v)
    pltpu.sync_copy(off_hbm, off_v)

    def compute_idx(p):
        for c in range(NUM_VECS):
            pv = pos_v[pl.ds(c * 16, 16)]
            ov = off_v[pl.ds(c * 16, 16)]
            idxs[p][pl.ds(c * 16, 16)] = plsc.load_gather(raws[p], [pv]) + ov

    def gather_copies(p):
        return [
            pltpu.make_async_copy(
                tab_hbm.at[idxs[p].at[pl.ds(j * CHUNK, CHUNK)]],
                rows[p].at[pl.ds(j * CHUNK, CHUNK)], sem_g[p])
            for j in range(NUM_CHUNKS)
        ]

    def fire_gather(p):
        for cp in gather_copies(p):
            cp.start()

    def wait_gather(p):
        for cp in gather_copies(p):
            cp.wait()

    def fire_raw(i_next, p):
        src = jnp.minimum(base + i_next, NUM_PAIRS - 1)
        pltpu.make_async_copy(raw_hbm.at[src], raws[p], sem_r[p]).start()

    def wait_raw(p):
        pltpu.make_async_copy(raw_hbm.at[0], raws[p], sem_r[p]).wait()

    def fire_out(i, p):
        pltpu.make_async_copy(rows[p], out_hbm.at[base + i], sem_o[p]).start()

    def wait_out(p):
        pltpu.make_async_copy(rows[p], out_hbm.at[0], sem_o[p]).wait()

    pltpu.sync_copy(raw_hbm.at[base], raws[0])
    compute_idx(0)
    fire_gather(0)
    pltpu.sync_copy(raw_hbm.at[base + 1], raws[1])
    compute_idx(1)
    fire_gather(1)
    fire_raw(2, 0)
    wait_gather(0)
    fire_out(0, 0)

    def step(i, p):
        wait_raw(p)
        compute_idx(p)
        fire_raw(i + 1, 1 - p)
        wait_out(p)
        fire_gather(p)
        wait_gather(1 - p)
        fire_out(i - 1, 1 - p)

    def body(k, carry):
        step(2 * k, 0)
        step(2 * k + 1, 1)
        return carry

    lax.fori_loop(1, PAIRS_PER_W // 2, body, 0)

    wait_raw(0)
    wait_gather(1)
    fire_out(PAIRS_PER_W - 1, 1)
    wait_out(0)
    wait_out(1)


@jax.jit
def _run(tab, raw, pos, off):
    mesh = plsc.VectorSubcoreMesh(core_axis_name="c", subcore_axis_name="s")
    f = functools.partial(
        pl.kernel,
        mesh=mesh,
        compiler_params=pltpu.CompilerParams(
            needs_layout_passes=False, use_tc_tiling_on_sc=False),
        out_type=jax.ShapeDtypeStruct(
            (NUM_PAIRS, ROWS_PER_PAIR, EMB_DIM), jnp.float32),
        scratch_types=[
            pltpu.VMEM((ROWS_PER_PAIR,), jnp.int32),
            pltpu.VMEM((ROWS_PER_PAIR,), jnp.int32),
            pltpu.VMEM((RAW_PER_PAIR,), jnp.int32),
            pltpu.VMEM((RAW_PER_PAIR,), jnp.int32),
            pltpu.VMEM((ROWS_PER_PAIR,), jnp.int32),
            pltpu.VMEM((ROWS_PER_PAIR,), jnp.int32),
            pltpu.VMEM((ROWS_PER_PAIR, EMB_DIM), jnp.float32),
            pltpu.VMEM((ROWS_PER_PAIR, EMB_DIM), jnp.float32),
            pltpu.SemaphoreType.DMA,
            pltpu.SemaphoreType.DMA,
            pltpu.SemaphoreType.DMA,
            pltpu.SemaphoreType.DMA,
            pltpu.SemaphoreType.DMA,
            pltpu.SemaphoreType.DMA,
        ],
    )(_sc_body)
    return f(tab, raw, pos, off)


def kernel(sequence_bucket_inputs, tables):
    raw = sequence_bucket_inputs.reshape(NUM_PAIRS, RAW_PER_PAIR)
    tab = tables.reshape(TIME_STEPS * NUM_SLOTS * NUM_EMB, EMB_DIM)
    out = _run(tab, raw, jnp.asarray(_POS), jnp.asarray(_OFF))
    return out.reshape(BATCH, TIME_STEPS, NUM_SLOTS * EMB_DIM)


# speedup vs baseline: 18.7503x; 2.4342x over previous
import functools

import jax
import jax.numpy as jnp
from jax import lax
from jax.experimental import pallas as pl
from jax.experimental.pallas import tpu as pltpu
from jax.experimental.pallas import tpu_sc as plsc

MAX_SLOT = 20
TIME_STEPS = 20
NUM_SLOTS = 18
NUM_EMB = 1002
EMB_DIM = 32
BATCH = 1024

NUM_WORKERS = 32
NUM_FACES = TIME_STEPS * NUM_SLOTS
LANES = 16
B_CHUNKS = BATCH // LANES
HALF = EMB_DIM // 2


def _sc_body(tab_hbm, raw_hbm, out_hbm, ids_v, face_v, stage_v, sem):
    wid = lax.axis_index("s") * 2 + lax.axis_index("c")
    f_lo = (wid * NUM_FACES) // NUM_WORKERS
    f_hi = ((wid + 1) * NUM_FACES) // NUM_WORKERS

    def face_body(f, carry):
        t = f // NUM_SLOTS
        j = f % NUM_SLOTS
        slot = j + 1 + (j >= 4).astype(jnp.int32)
        pltpu.sync_copy(raw_hbm.at[t, slot], ids_v)
        pltpu.sync_copy(tab_hbm.at[t, j], face_v)

        def half_body(h, carry2):
            def chunk_body(bc, carry3):
                ev = ids_v[pl.ds(bc * LANES, LANES)]
                for d in range(HALF):
                    dv = jnp.full((LANES,), d, jnp.int32) + h * HALF
                    vec = plsc.load_gather(face_v, [dv, ev])
                    stage_v[d, pl.ds(bc * LANES, LANES)] = vec
                return carry3

            lax.fori_loop(0, B_CHUNKS, chunk_body, 0)
            row0 = pl.multiple_of(EMB_DIM * j + HALF * h, HALF)
            pltpu.sync_copy(stage_v, out_hbm.at[t, pl.ds(row0, HALF), :])
            return carry2

        lax.fori_loop(0, 2, half_body, 0)
        return carry

    lax.fori_loop(f_lo, f_hi, face_body, 0)


@jax.jit
def _run(tab, raw):
    mesh = plsc.VectorSubcoreMesh(core_axis_name="c", subcore_axis_name="s")
    f = functools.partial(
        pl.kernel,
        mesh=mesh,
        compiler_params=pltpu.CompilerParams(needs_layout_passes=False),
        out_type=jax.ShapeDtypeStruct(
            (TIME_STEPS, NUM_SLOTS * EMB_DIM, BATCH), jnp.float32),
        scratch_types=[
            pltpu.VMEM((BATCH,), jnp.int32),
            pltpu.VMEM((EMB_DIM, NUM_EMB), jnp.float32),
            pltpu.VMEM((HALF, BATCH), jnp.float32),
            pltpu.SemaphoreType.DMA,
        ],
    )(_sc_body)
    return f(tab, raw)


def kernel(sequence_bucket_inputs, tables):
    tab_t = tables.transpose(0, 1, 3, 2)
    raw_t = sequence_bucket_inputs.transpose(1, 2, 0)
    out = _run(tab_t, raw_t)
    return out.transpose(2, 0, 1)


# speedup vs baseline: 24.8534x; 1.3255x over previous
import functools

import jax
import jax.numpy as jnp
from jax import lax
from jax.experimental import pallas as pl
from jax.experimental.pallas import tpu as pltpu
from jax.experimental.pallas import tpu_sc as plsc

MAX_SLOT = 20
TIME_STEPS = 20
NUM_SLOTS = 18
NUM_EMB = 1002
EMB_DIM = 32
BATCH = 1024

NUM_WORKERS = 32
NUM_FACES = TIME_STEPS * NUM_SLOTS
LANES = 16
B_CHUNKS = BATCH // LANES
HALF = EMB_DIM // 2


def _sc_body(tab_hbm, raw_hbm, out_hbm,
             ids_a, ids_b, face_a, face_b, stage0, stage1,
             sem_i, sem_o0, sem_o1):
    wid = lax.axis_index("s") * 2 + lax.axis_index("c")
    f_lo = (wid * NUM_FACES) // NUM_WORKERS
    f_hi = ((wid + 1) * NUM_FACES) // NUM_WORKERS
    idss, faces = (ids_a, ids_b), (face_a, face_b)
    stages, sem_o = (stage0, stage1), (sem_o0, sem_o1)

    def in_copies(f, p):
        t = f // NUM_SLOTS
        j = f % NUM_SLOTS
        slot = j + 1 + (j >= 4).astype(jnp.int32)
        return (pltpu.make_async_copy(raw_hbm.at[t, slot], idss[p], sem_i),
                pltpu.make_async_copy(tab_hbm.at[t, j], faces[p], sem_i))

    def fire_in(f, p):
        for cp in in_copies(f, p):
            cp.start()

    def wait_in(f, p):
        for cp in in_copies(f, p):
            cp.wait()

    def out_copy(t, j, h):
        row0 = pl.multiple_of(EMB_DIM * j + HALF * h, HALF)
        return pltpu.make_async_copy(
            stages[h], out_hbm.at[t, pl.ds(row0, HALF), :], sem_o[h])

    def face_compute(f, r, p):
        t = f // NUM_SLOTS
        j = f % NUM_SLOTS
        wait_in(f, p)

        @pl.when(f + 1 < f_hi)
        def _prefetch():
            fire_in(f + 1, 1 - p)

        for h in range(2):
            @pl.when(r >= 1)
            def _drain():
                out_copy(0, 0, h).wait()

            def chunk_body(bc, carry, h=h, p=p):
                ev = idss[p][pl.ds(bc * LANES, LANES)]
                for d in range(HALF):
                    dv = jnp.full((LANES,), h * HALF + d, jnp.int32)
                    stages[h][d, pl.ds(bc * LANES, LANES)] = (
                        plsc.load_gather(faces[p], [dv, ev]))
                return carry

            lax.fori_loop(0, B_CHUNKS, chunk_body, 0)
            out_copy(t, j, h).start()

    fire_in(f_lo, 0)

    def face_body(r, carry):
        f = f_lo + r

        @pl.when(r % 2 == 0)
        def _even():
            face_compute(f, r, 0)

        @pl.when(r % 2 == 1)
        def _odd():
            face_compute(f, r, 1)

        return carry

    lax.fori_loop(0, f_hi - f_lo, face_body, 0)
    out_copy(0, 0, 0).wait()
    out_copy(0, 0, 1).wait()


@jax.jit
def _run(tab, raw):
    mesh = plsc.VectorSubcoreMesh(core_axis_name="c", subcore_axis_name="s")
    f = functools.partial(
        pl.kernel,
        mesh=mesh,
        compiler_params=pltpu.CompilerParams(needs_layout_passes=False),
        out_type=jax.ShapeDtypeStruct(
            (TIME_STEPS, NUM_SLOTS * EMB_DIM, BATCH), jnp.float32),
        scratch_types=[
            pltpu.VMEM((BATCH,), jnp.int32),
            pltpu.VMEM((BATCH,), jnp.int32),
            pltpu.VMEM((EMB_DIM, NUM_EMB), jnp.float32),
            pltpu.VMEM((EMB_DIM, NUM_EMB), jnp.float32),
            pltpu.VMEM((HALF, BATCH), jnp.float32),
            pltpu.VMEM((HALF, BATCH), jnp.float32),
            pltpu.SemaphoreType.DMA,
            pltpu.SemaphoreType.DMA,
            pltpu.SemaphoreType.DMA,
        ],
    )(_sc_body)
    return f(tab, raw)


def kernel(sequence_bucket_inputs, tables):
    tab_t = tables.transpose(0, 1, 3, 2)
    raw_t = sequence_bucket_inputs.transpose(1, 2, 0)
    out = _run(tab_t, raw_t)
    return out.transpose(2, 0, 1)


# speedup vs baseline: 41.9500x; 1.6879x over previous
import functools

import jax
import jax.numpy as jnp
from jax import lax
from jax.experimental import pallas as pl
from jax.experimental.pallas import tpu as pltpu
from jax.experimental.pallas import tpu_sc as plsc

MAX_SLOT = 20
TIME_STEPS = 20
NUM_SLOTS = 18
NUM_EMB = 1002
EMB_DIM = 32
BATCH = 1024

NUM_WORKERS = 32
NUM_FACES = TIME_STEPS * NUM_SLOTS
LANES = 16
B_CHUNKS = BATCH // LANES
HALF = EMB_DIM // 2


def _sc_body(tab_hbm, raw_hbm, out_hbm,
             ids_a, ids_b, face_a, face_b, stage0, stage1,
             sem_i, sem_o0, sem_o1):
    wid = lax.axis_index("s") * 2 + lax.axis_index("c")
    f_lo = (wid * NUM_FACES) // NUM_WORKERS
    f_hi = ((wid + 1) * NUM_FACES) // NUM_WORKERS
    idss, faces = (ids_a, ids_b), (face_a, face_b)
    stages, sem_o = (stage0, stage1), (sem_o0, sem_o1)

    def in_copies(f, p):
        t = f // NUM_SLOTS
        j = f % NUM_SLOTS
        slot = j + 1 + (j >= 4).astype(jnp.int32)
        return (pltpu.make_async_copy(raw_hbm.at[t, slot], idss[p], sem_i),
                pltpu.make_async_copy(tab_hbm.at[t, j], faces[p], sem_i))

    def fire_in(f, p):
        for cp in in_copies(f, p):
            cp.start()

    def wait_in(f, p):
        for cp in in_copies(f, p):
            cp.wait()

    def out_copy(t, j, h):
        row0 = pl.multiple_of(EMB_DIM * j + HALF * h, HALF)
        return pltpu.make_async_copy(
            stages[h], out_hbm.at[t, pl.ds(row0, HALF), :], sem_o[h])

    def face_compute(f, r, p):
        t = f // NUM_SLOTS
        j = f % NUM_SLOTS
        wait_in(f, p)

        @pl.when(f + 1 < f_hi)
        def _prefetch():
            fire_in(f + 1, 1 - p)

        for h in range(2):
            @pl.when(r >= 1)
            def _drain():
                out_copy(0, 0, h).wait()

            def chunk_body(bc, carry, h=h, p=p):
                ev = idss[p][pl.ds(bc * LANES, LANES)]
                vals = [
                    plsc.load_gather(
                        faces[p],
                        [jnp.full((LANES,), h * HALF + d, jnp.int32), ev])
                    for d in range(HALF)
                ]
                for d in range(HALF):
                    stages[h][d, pl.ds(bc * LANES, LANES)] = vals[d]
                return carry

            lax.fori_loop(0, B_CHUNKS, chunk_body, 0)
            out_copy(t, j, h).start()

    fire_in(f_lo, 0)

    def face_body(r, carry):
        f = f_lo + r

        @pl.when(r % 2 == 0)
        def _even():
            face_compute(f, r, 0)

        @pl.when(r % 2 == 1)
        def _odd():
            face_compute(f, r, 1)

        return carry

    lax.fori_loop(0, f_hi - f_lo, face_body, 0)
    out_copy(0, 0, 0).wait()
    out_copy(0, 0, 1).wait()


@jax.jit
def _run(tab, raw):
    mesh = plsc.VectorSubcoreMesh(core_axis_name="c", subcore_axis_name="s")
    f = functools.partial(
        pl.kernel,
        mesh=mesh,
        compiler_params=pltpu.CompilerParams(needs_layout_passes=False),
        out_type=jax.ShapeDtypeStruct(
            (TIME_STEPS, NUM_SLOTS * EMB_DIM, BATCH), jnp.float32),
        scratch_types=[
            pltpu.VMEM((BATCH,), jnp.int32),
            pltpu.VMEM((BATCH,), jnp.int32),
            pltpu.VMEM((EMB_DIM, NUM_EMB), jnp.float32),
            pltpu.VMEM((EMB_DIM, NUM_EMB), jnp.float32),
            pltpu.VMEM((HALF, BATCH), jnp.float32),
            pltpu.VMEM((HALF, BATCH), jnp.float32),
            pltpu.SemaphoreType.DMA,
            pltpu.SemaphoreType.DMA,
            pltpu.SemaphoreType.DMA,
        ],
    )(_sc_body)
    return f(tab, raw)


def kernel(sequence_bucket_inputs, tables):
    tab_t = tables.transpose(0, 1, 3, 2)
    raw_t = sequence_bucket_inputs.transpose(1, 2, 0)
    out = _run(tab_t, raw_t)
    return out.transpose(2, 0, 1)


# speedup vs baseline: 45.2699x; 1.0791x over previous
import functools

import jax
import jax.numpy as jnp
from jax import lax
from jax.experimental import pallas as pl
from jax.experimental.pallas import tpu as pltpu
from jax.experimental.pallas import tpu_sc as plsc

MAX_SLOT = 20
TIME_STEPS = 20
NUM_SLOTS = 18
NUM_EMB = 1002
EMB_DIM = 32
BATCH = 1024

NUM_WORKERS = 32
NUM_FACES = TIME_STEPS * NUM_SLOTS
LANES = 16
B_CHUNKS = BATCH // LANES
HALF = EMB_DIM // 2


NUM_UNITS = NUM_FACES * 2


def _sc_body(tab_hbm, raw_hbm, out_hbm,
             ids_a, ids_b, face_a, face_b, stage_a, stage_b,
             sem_i, sem_oa, sem_ob):
    wid = lax.axis_index("s") * 2 + lax.axis_index("c")
    u_lo = (wid * NUM_UNITS) // NUM_WORKERS
    u_hi = ((wid + 1) * NUM_UNITS) // NUM_WORKERS
    idss, faces = (ids_a, ids_b), (face_a, face_b)
    stages, sem_o = (stage_a, stage_b), (sem_oa, sem_ob)

    def unit_tjsh(u):
        f = u // 2
        hh = u % 2
        j = f % NUM_SLOTS
        slot = j + 1 + (j >= 4).astype(jnp.int32)
        return f // NUM_SLOTS, j, slot, hh

    def in_copies(u, p):
        t, j, slot, hh = unit_tjsh(u)
        row = pl.multiple_of(HALF * hh, HALF)
        return (pltpu.make_async_copy(raw_hbm.at[t, slot], idss[p], sem_i),
                pltpu.make_async_copy(
                    tab_hbm.at[t, j, pl.ds(row, HALF), :], faces[p], sem_i))

    def fire_in(u, p):
        for cp in in_copies(u, p):
            cp.start()

    def wait_in(u, p):
        for cp in in_copies(u, p):
            cp.wait()

    def out_copy(u, p):
        t, j, _, hh = unit_tjsh(u)
        row0 = pl.multiple_of(EMB_DIM * j + HALF * hh, HALF)
        return pltpu.make_async_copy(
            stages[p], out_hbm.at[t, pl.ds(row0, HALF), :], sem_o[p])

    def unit_compute(u, r, p):
        wait_in(u, p)

        @pl.when(u + 1 < u_hi)
        def _prefetch():
            fire_in(u + 1, 1 - p)

        @pl.when(r >= 2)
        def _drain():
            out_copy(u, p).wait()

        def chunk_body(bc, carry, p=p):
            ev = idss[p][pl.ds(bc * LANES, LANES)]
            vals = [
                plsc.load_gather(
                    faces[p], [jnp.full((LANES,), d, jnp.int32), ev])
                for d in range(HALF)
            ]
            for d in range(HALF):
                stages[p][d, pl.ds(bc * LANES, LANES)] = vals[d]
            return carry

        lax.fori_loop(0, B_CHUNKS, chunk_body, 0)
        out_copy(u, p).start()

    fire_in(u_lo, 0)

    def unit_body(r, carry):
        u = u_lo + r

        @pl.when(r % 2 == 0)
        def _even():
            unit_compute(u, r, 0)

        @pl.when(r % 2 == 1)
        def _odd():
            unit_compute(u, r, 1)

        return carry

    lax.fori_loop(0, u_hi - u_lo, unit_body, 0)
    out_copy(u_lo, 0).wait()
    out_copy(u_lo, 1).wait()


@jax.jit
def _run(tab, raw):
    mesh = plsc.VectorSubcoreMesh(core_axis_name="c", subcore_axis_name="s")
    f = functools.partial(
        pl.kernel,
        mesh=mesh,
        compiler_params=pltpu.CompilerParams(needs_layout_passes=False),
        out_type=jax.ShapeDtypeStruct(
            (TIME_STEPS, NUM_SLOTS * EMB_DIM, BATCH), jnp.float32),
        scratch_types=[
            pltpu.VMEM((BATCH,), jnp.int32),
            pltpu.VMEM((BATCH,), jnp.int32),
            pltpu.VMEM((HALF, NUM_EMB), jnp.float32),
            pltpu.VMEM((HALF, NUM_EMB), jnp.float32),
            pltpu.VMEM((HALF, BATCH), jnp.float32),
            pltpu.VMEM((HALF, BATCH), jnp.float32),
            pltpu.SemaphoreType.DMA,
            pltpu.SemaphoreType.DMA,
            pltpu.SemaphoreType.DMA,
        ],
    )(_sc_body)
    return f(tab, raw)


def kernel(sequence_bucket_inputs, tables):
    tab_t = tables.transpose(0, 1, 3, 2)
    raw_t = sequence_bucket_inputs.transpose(1, 2, 0)
    out = _run(tab_t, raw_t)
    return out.transpose(2, 0, 1)
